# speedup vs baseline: 6.3517x; 6.3517x over previous
import functools

import jax
import jax.numpy as jnp
from jax import lax
from jax.experimental import pallas as pl
from jax.experimental.pallas import tpu as pltpu
from jax.experimental.pallas import tpu_sc as plsc

N_NODES = 100000
FEAT = 128
B = 50000
S = 10
SLOTS = S + 1

NC = 2
NS = 16
NW = NC * NS

B_PAD = 51200
B_PER_W = B_PAD // NW
C = 64
NCHUNK = B_PER_W // C


def _sc_body(idx_hbm, feat_hbm, self_hbm, nsum_hbm,
             idx_v, gbuf, didx_v, sacc, gsem):
    c = lax.axis_index("c")
    s = lax.axis_index("s")
    wid = s * NC + c

    for k in range(C // 16):
        didx_v[pl.ds(16 * k, 16)] = (
            s * C + 16 * k + lax.iota(jnp.int32, 16)
        )

    def chunk_body(ci, carry):
        base = wid * B_PER_W + ci * C
        pltpu.sync_copy(idx_hbm.at[wid, ci], idx_v)
        cps = [
            pltpu.async_copy(feat_hbm.at[idx_v.at[t]],
                             gbuf.at[pl.ds(t * C, C)], gsem)
            for t in range(SLOTS)
        ]
        for cp in cps:
            cp.wait()
        pltpu.sync_copy(gbuf.at[pl.ds(0, C)], self_hbm.at[pl.ds(base, C)])
        pltpu.sync_copy(gbuf.at[pl.ds(C, C)], sacc.at[pl.ds(s * C, C)])
        for t in range(2, SLOTS):
            pltpu.sync_copy(gbuf.at[pl.ds(t * C, C)], sacc.at[didx_v],
                            add=True)
        pltpu.sync_copy(sacc.at[pl.ds(s * C, C)], nsum_hbm.at[pl.ds(base, C)])
        return carry

    lax.fori_loop(0, NCHUNK, chunk_body, 0)


def _tc_body(self_ref, nsum_ref, ws_ref, wn_ref, w1s_ref, w1n_ref, o_ref):
    xs = self_ref[...]
    xn = nsum_ref[...] * jnp.float32(1.0 / S)
    comb = jnp.tanh(
        jnp.dot(xs, ws_ref[...], preferred_element_type=jnp.float32)
        + jnp.dot(xn, wn_ref[...], preferred_element_type=jnp.float32)
    )
    att = jax.nn.sigmoid(
        jnp.dot(xs, w1s_ref[...], preferred_element_type=jnp.float32)
        + jnp.dot(xn, w1n_ref[...], preferred_element_type=jnp.float32)
    )
    o_ref[...] = att * (comb + jnp.tanh(xn))


_sc_gather = pl.kernel(
    _sc_body,
    out_type=[
        jax.ShapeDtypeStruct((B_PAD, FEAT), jnp.float32),
        jax.ShapeDtypeStruct((B_PAD, FEAT), jnp.float32),
    ],
    mesh=plsc.VectorSubcoreMesh(core_axis_name="c", subcore_axis_name="s"),
    scratch_types=[
        pltpu.VMEM((SLOTS, C), jnp.int32),
        pltpu.VMEM((SLOTS * C, FEAT), jnp.float32),
        pltpu.VMEM((C,), jnp.int32),
        pltpu.VMEM_SHARED((NS * C, FEAT), jnp.float32),
        pltpu.SemaphoreType.DMA,
    ],
)

_TC_R = 2000


@jax.jit
def _run(nodes, neigh_idx, features, weight, weight1):
    pad = B_PAD - B
    nodes_p = jnp.concatenate(
        [nodes, jnp.arange(pad, dtype=jnp.int32) % N_NODES])
    neigh_p = jnp.concatenate(
        [neigh_idx,
         (jnp.arange(pad * S, dtype=jnp.int32) % N_NODES).reshape(pad, S)])
    self_i = nodes_p.reshape(NW, NCHUNK, 1, C)
    neigh_i = neigh_p.reshape(NW, NCHUNK, C, S).transpose(0, 1, 3, 2)
    idx_all = jnp.concatenate([self_i, neigh_i], axis=2)

    self_f, nsum_f = _sc_gather(idx_all, features)

    ws = weight[:, :FEAT].T
    wn = weight[:, FEAT:].T
    w1s = weight1[:, :FEAT].T
    w1n = weight1[:, FEAT:].T

    wspec = pl.BlockSpec((FEAT, FEAT), lambda i: (0, 0))
    out = pl.pallas_call(
        _tc_body,
        grid=(B // _TC_R,),
        in_specs=[
            pl.BlockSpec((_TC_R, FEAT), lambda i: (i, 0)),
            pl.BlockSpec((_TC_R, FEAT), lambda i: (i, 0)),
            wspec, wspec, wspec, wspec,
        ],
        out_specs=pl.BlockSpec((_TC_R, FEAT), lambda i: (i, 0)),
        out_shape=jax.ShapeDtypeStruct((B, FEAT), jnp.float32),
    )(self_f, nsum_f, ws, wn, w1s, w1n)
    return out


def kernel(nodes, neigh_idx, features, weight, weight1):
    return _run(nodes, neigh_idx, features, weight, weight1)
